# baseline (speedup 1.0000x reference)
import jax
import jax.numpy as jnp
from jax.experimental import pallas as pl
from jax.experimental.pallas import tpu as pltpu

B = 16384
D = 128
CHR = 4096
NCHUNK = B // CHR
NBUF = 4


def _row_dot_kernel(gu_hbm, gi_hbm, bu_ref, bi_ref, mu_ref, out_ref,
                    gub, gib, sg, si):
    def copy_in(c):
        b = c % NBUF
        pltpu.make_async_copy(
            gu_hbm.at[pl.ds(c * CHR, CHR)], gub.at[b], sg.at[b]).start()
        pltpu.make_async_copy(
            gi_hbm.at[pl.ds(c * CHR, CHR)], gib.at[b], si.at[b]).start()

    for c in range(NBUF):
        copy_in(c)

    ones = jnp.ones((1, D), dtype=jnp.float32)
    mu = mu_ref[0, 0]
    for c in range(NCHUNK):
        b = c % NBUF
        pltpu.make_async_copy(
            gu_hbm.at[pl.ds(c * CHR, CHR)], gub.at[b], sg.at[b]).wait()
        pltpu.make_async_copy(
            gi_hbm.at[pl.ds(c * CHR, CHR)], gib.at[b], si.at[b]).wait()
        p = gub[b] * gib[b]
        s = jax.lax.dot_general(
            ones, p, (((1,), (1,)), ((), ())),
            preferred_element_type=jnp.float32,
        )
        sl = pl.ds(c * CHR, CHR)
        out_ref[sl] = s.reshape(CHR) + bu_ref[sl] + bi_ref[sl] + mu
        if c + NBUF < NCHUNK:
            copy_in(c + NBUF)


def kernel(gu, gi, bu, bi, Mu):
    bu_f = bu.reshape(B)
    bi_f = bi.reshape(B)
    out = pl.pallas_call(
        _row_dot_kernel,
        in_specs=[
            pl.BlockSpec(memory_space=pltpu.HBM),
            pl.BlockSpec(memory_space=pltpu.HBM),
            pl.BlockSpec(memory_space=pltpu.VMEM),
            pl.BlockSpec(memory_space=pltpu.VMEM),
            pl.BlockSpec(memory_space=pltpu.VMEM),
        ],
        out_specs=pl.BlockSpec(memory_space=pltpu.VMEM),
        out_shape=jax.ShapeDtypeStruct((B,), jnp.float32),
        scratch_shapes=[
            pltpu.VMEM((NBUF, CHR, D), jnp.float32),
            pltpu.VMEM((NBUF, CHR, D), jnp.float32),
            pltpu.SemaphoreType.DMA((NBUF,)),
            pltpu.SemaphoreType.DMA((NBUF,)),
        ],
    )(gu, gi, bu_f, bi_f, Mu)
    return out


# speedup vs baseline: 1.0824x; 1.0824x over previous
import jax
import jax.numpy as jnp
from jax.experimental import pallas as pl

B = 16384
D = 128
BLK = 8192


def _row_dot_kernel(gu_ref, gi_ref, bu_ref, bi_ref, mu_ref, out_ref):
    p = gu_ref[...] * gi_ref[...]
    ones = jnp.ones((1, D), dtype=jnp.float32)
    s = jax.lax.dot_general(
        ones, p, (((1,), (1,)), ((), ())),
        preferred_element_type=jnp.float32,
    )
    out_ref[...] = s.reshape(BLK) + bu_ref[...] + bi_ref[...] + mu_ref[0, 0]


def kernel(gu, gi, bu, bi, Mu):
    bu_f = bu.reshape(B)
    bi_f = bi.reshape(B)
    grid = (B // BLK,)
    out = pl.pallas_call(
        _row_dot_kernel,
        grid=grid,
        in_specs=[
            pl.BlockSpec((BLK, D), lambda i: (i, 0)),
            pl.BlockSpec((BLK, D), lambda i: (i, 0)),
            pl.BlockSpec((BLK,), lambda i: (i,)),
            pl.BlockSpec((BLK,), lambda i: (i,)),
            pl.BlockSpec((1, 1), lambda i: (0, 0)),
        ],
        out_specs=pl.BlockSpec((BLK,), lambda i: (i,)),
        out_shape=jax.ShapeDtypeStruct((B,), jnp.float32),
    )(gu, gi, bu_f, bi_f, Mu)
    return out
